# baseline (speedup 1.0000x reference)
import functools

import jax
import jax.numpy as jnp
from jax import lax
from jax.experimental import pallas as pl
from jax.experimental.pallas import tpu as pltpu
from jax.experimental.pallas import tpu_sc as plsc

NC = 2
NS = 16
NW = NC * NS
CHUNK = 128
LANES = 16
MAXCH = 129


def _sc_update(B, D, R):
    mesh = plsc.VectorSubcoreMesh(core_axis_name="c", subcore_axis_name="s")

    @functools.partial(
        pl.kernel,
        out_type=jax.ShapeDtypeStruct((R * NW, D), jnp.float32),
        mesh=mesh,
        compiler_params=pltpu.CompilerParams(
            needs_layout_passes=False, use_tc_tiling_on_sc=False),
        scratch_types=[
            pltpu.VMEM((CHUNK,), jnp.int32),
            pltpu.VMEM((CHUNK,), jnp.int32),
            pltpu.VMEM((CHUNK, D), jnp.float32),
            pltpu.VMEM((CHUNK, D), jnp.float32),
            pltpu.VMEM((LANES,), jnp.float32),
            pltpu.VMEM((LANES,), jnp.int32),
            pltpu.VMEM((LANES,), jnp.int32),
            pltpu.SemaphoreType.DMA,
        ],
    )
    def body(idx_hbm, pos_hbm, start_hbm, end_hbm, mem_hbm, mom_hbm,
             memory_hbm, table, idxv, posv, oldv, newv, momv, startv, endv,
             sem):
        c = lax.axis_index("c")
        s = lax.axis_index("s")
        w = s * NC + c
        cp = pltpu.async_copy(memory_hbm.at[pl.ds(w * R, R)],
                              table.at[pl.ds(w * R, R)], sem)
        pltpu.sync_copy(mom_hbm, momv)
        pltpu.sync_copy(start_hbm.at[w], startv)
        pltpu.sync_copy(end_hbm.at[w], endv)
        mval = momv[...]
        one_m = 1.0 - mval
        start = jnp.max(startv[...])
        end = jnp.max(endv[...])
        cp.wait()

        @pl.loop(0, MAXCH)
        def _chunk(j):
            base = pl.multiple_of(start + j * CHUNK, 8)

            @pl.when(base < end)
            def _():
                pltpu.sync_copy(idx_hbm.at[pl.ds(base, CHUNK)], idxv)
                pltpu.sync_copy(pos_hbm.at[pl.ds(base, CHUNK)], posv)
                pltpu.async_copy(memory_hbm.at[idxv], oldv, sem).wait()
                pltpu.async_copy(mem_hbm.at[posv], newv, sem).wait()

                @pl.loop(0, CHUNK)
                def _row(r):
                    acc = jnp.zeros((LANES,), jnp.float32)
                    for k in range(D // LANES):
                        o = oldv[r, pl.ds(k * LANES, LANES)]
                        n = newv[r, pl.ds(k * LANES, LANES)]
                        u = o * one_m + n * mval
                        oldv[r, pl.ds(k * LANES, LANES)] = u
                        acc = acc + u * u
                    ssum = jnp.sum(acc)
                    sv = lax.broadcast_in_dim(ssum, (LANES,), ())
                    iv = plsc.bitcast(sv, jnp.int32)
                    iv = jnp.int32(0x5F3759DF) - lax.shift_right_logical(iv, 1)
                    y = plsc.bitcast(iv, jnp.float32)
                    for _ in range(3):
                        y = y * (1.5 - 0.5 * sv * y * y)
                    for k in range(D // LANES):
                        oldv[r, pl.ds(k * LANES, LANES)] = (
                            oldv[r, pl.ds(k * LANES, LANES)] * y)

                pltpu.async_copy(oldv, table.at[idxv], sem).wait()

    return body


def kernel(mem, momentum, ind, time, memory):
    mem2 = mem.reshape(mem.shape[0], -1)
    B, D = mem2.shape
    L = memory.shape[0]
    ind32 = ind.astype(jnp.int32)
    R = L // NW

    iota = jnp.arange(B, dtype=jnp.int32)
    pos_table = jnp.zeros((L,), jnp.int32).at[ind32].set(iota)
    winner_pos = pos_table[ind32]

    owner = ind32 // R
    order = jnp.argsort(owner, stable=True)
    ind_s = ind32[order]
    pos_s = winner_pos[order]
    counts = jnp.zeros((NW,), jnp.int32).at[owner].add(1)
    ends = jnp.cumsum(counts).astype(jnp.int32)
    starts = (ends - counts) & -8

    ind_p = jnp.concatenate([ind_s, jnp.broadcast_to(ind_s[-1:], (CHUNK,))])
    pos_p = jnp.concatenate([pos_s, jnp.broadcast_to(pos_s[-1:], (CHUNK,))])
    start16 = jnp.broadcast_to(starts[:, None], (NW, LANES))
    end16 = jnp.broadcast_to(ends[:, None], (NW, LANES))
    mom16 = jnp.full((LANES,), momentum, jnp.float32)

    return _sc_update(B, D, R)(
        ind_p, pos_p, start16, end16, mem2, mom16, memory)
